# baseline (speedup 1.0000x reference)
import jax
import jax.numpy as jnp
from jax.experimental import pallas as pl
from jax.experimental.pallas import tpu as pltpu

_IN_CH = 96
_OUT_CH = 192
_K = 3
_STRIDE = 2
_PAD = 1
_TABLE = 16
_NH = 4
_M = 9
_U = 0.99
_R = 2.5
_B, _H, _W = 2, 224, 224
_HO = _WO = 112
_RB = 16
_NRB = _HO // _RB
_IRB = 2 * _RB
_ANCH = 32
_PCH = _OUT_CH - _ANCH
_OCB = 32
_NOCB = _PCH // _OCB


def _conv_body(wref, xm, xh, oref):
    k = pl.program_id(1)

    xblk = xm[...]
    even = jnp.concatenate([xblk[:, 2 * j:2 * j + 1, :] for j in range(_RB)],
                           axis=1)
    odd = jnp.concatenate([xblk[:, 2 * j + 1:2 * j + 2, :] for j in range(_RB)],
                          axis=1)
    halo = xh[:, 7:8, :]
    halo = jnp.where(k == 0, jnp.zeros_like(halo), halo)
    oprev = jnp.concatenate([halo, odd[:, :-1, :]], axis=1)

    def mm(w, rows):
        return jax.lax.dot_general(
            w, rows.reshape(_IN_CH, _RB * _W),
            (((1,), (0,)), ((), ())),
            preferred_element_type=jnp.float32)

    w = wref[...]
    t0 = mm(w[0, 0], oprev) + mm(w[1, 0], even) + mm(w[2, 0], odd)
    t1 = mm(w[0, 1], oprev) + mm(w[1, 1], even) + mm(w[2, 1], odd)
    t2 = mm(w[0, 2], oprev) + mm(w[1, 2], even) + mm(w[2, 2], odd)
    t0 = t0.reshape(_OCB, _RB, _W)
    t1 = t1.reshape(_OCB, _RB, _W)
    t2 = t2.reshape(_OCB, _RB, _W)
    rolled = jnp.concatenate(
        [jnp.zeros((_OCB, _RB, 2), jnp.float32), t0[:, :, :-2]], axis=2)
    u = t2 + rolled
    cat = jnp.concatenate([t1, u], axis=2).reshape(_OCB * _RB, 2 * _W)
    src = jax.lax.broadcasted_iota(jnp.int32, (2 * _W, _WO), 0)
    dst = jax.lax.broadcasted_iota(jnp.int32, (2 * _W, _WO), 1)
    sel = ((src == 2 * dst) | (src == 2 * dst + 1 + _W)).astype(jnp.float32)
    oref[...] = jax.lax.dot_general(
        cat, sel, (((1,), (0,)), ((), ())),
        preferred_element_type=jnp.float32).reshape(_OCB, _RB, _WO)


def _conv_pallas(wt, x):
    return pl.pallas_call(
        _conv_body,
        grid=(_B, _NRB, _NOCB),
        in_specs=[
            pl.BlockSpec((_K, _K, _OCB, _IN_CH), lambda b, k, o: (0, 0, o, 0)),
            pl.BlockSpec((None, _IN_CH, _IRB, _W), lambda b, k, o: (b, 0, k, 0)),
            pl.BlockSpec((None, _IN_CH, 8, _W),
                         lambda b, k, o: (b, 0, jnp.maximum(_IRB // 8 * k - 1, 0), 0)),
        ],
        out_specs=pl.BlockSpec((None, _OCB, _RB, _WO), lambda b, k, o: (b, o, k, 0)),
        out_shape=jax.ShapeDtypeStruct((_B, _PCH, _HO, _WO), jnp.float32),
        compiler_params=pltpu.CompilerParams(
            dimension_semantics=("parallel", "arbitrary", "arbitrary")),
    )(wt, x, x)


def _active_mask(x, weight, hash_a, hash_b):
    w_flat = weight.reshape(_OUT_CH, -1)
    denom = jnp.linalg.norm(w_flat, axis=1).max()
    w_u = _U * w_flat / denom
    norms = jnp.linalg.norm(w_u, axis=1, keepdims=True)
    powers = jnp.concatenate([norms ** (2 ** (i + 1)) for i in range(_M)], axis=1)
    halves = jnp.full((_OUT_CH, _M), 0.5, dtype=w_u.dtype)
    w_pq = jnp.concatenate([w_u, powers, halves], axis=1)
    k_proj = w_pq @ hash_a.T + hash_b[None, :]
    k_idx = jnp.abs(jnp.mod(jnp.floor(k_proj / _R).astype(jnp.int32), _TABLE))

    x_u = _U * x / denom
    q_chan = jnp.full((_B, 1, _H, _W), 0.5, dtype=x.dtype)
    p_chan = jnp.broadcast_to(
        (jnp.linalg.norm(x_u.reshape(_B, -1), axis=1) ** 2).reshape(_B, 1, 1, 1),
        (_B, 1, _H, _W)).astype(x.dtype)
    x_aug = jnp.concatenate([x_u, q_chan, p_chan], axis=1)
    hk = hash_a.reshape(_NH, _IN_CH + 2, _K, _K)
    dotted = jax.lax.conv_general_dilated(
        x_aug, hk, window_strides=(_STRIDE, _STRIDE),
        padding=((_PAD, _PAD), (_PAD, _PAD)),
        rhs_dilation=(1, 1),
        dimension_numbers=('NCHW', 'OIHW', 'NCHW'))
    bucket = jnp.abs(jnp.mod(
        jnp.floor((dotted + hash_b.reshape(1, -1, 1, 1)) / _R).astype(jnp.int32),
        _TABLE))
    counts = jnp.stack([jnp.bincount(bucket[:, h].ravel(), length=_TABLE)
                        for h in range(_NH)])
    best = jnp.argmax(counts, axis=1)
    return jnp.any(k_idx == best[None, :], axis=1)


def kernel(x, weight, hash_a, hash_b):
    active = _active_mask(x, weight, hash_a, hash_b)
    scale = jnp.asarray(_NH / _TABLE, dtype=x.dtype)
    m = active.astype(x.dtype) * scale
    anchor = jax.lax.conv_general_dilated(
        x, weight[:_ANCH], window_strides=(_STRIDE, _STRIDE),
        padding=((_PAD, _PAD), (_PAD, _PAD)),
        rhs_dilation=(1, 1), dimension_numbers=('NCHW', 'OIHW', 'NCHW'))
    anchor = anchor * m[:_ANCH][None, :, None, None]
    w_eff = weight[_ANCH:] * m[_ANCH:][:, None, None, None]
    wt = jnp.transpose(w_eff, (2, 3, 0, 1))
    rest = _conv_pallas(wt, x)
    return jnp.concatenate([anchor, rest], axis=1)


# speedup vs baseline: 1.0005x; 1.0005x over previous
import jax
import jax.numpy as jnp
from jax.experimental import pallas as pl
from jax.experimental.pallas import tpu as pltpu

_IN_CH = 96
_OUT_CH = 192
_K = 3
_STRIDE = 2
_PAD = 1
_TABLE = 16
_NH = 4
_M = 9
_U = 0.99
_R = 2.5
_B, _H, _W = 2, 224, 224
_HO = _WO = 112
_RB = 16
_NRB = _HO // _RB
_IRB = 2 * _RB
_ANCH = 32
_PCH = _OUT_CH - _ANCH
_OCB = 32
_NOCB = _PCH // _OCB


def _conv_body(wref, xm, xh, oref):
    k = pl.program_id(1)

    xblk = xm[...]
    even = jnp.concatenate([xblk[:, 2 * j:2 * j + 1, :] for j in range(_RB)],
                           axis=1)
    odd = jnp.concatenate([xblk[:, 2 * j + 1:2 * j + 2, :] for j in range(_RB)],
                          axis=1)
    halo = xh[:, 7:8, :]
    halo = jnp.where(k == 0, jnp.zeros_like(halo), halo)
    oprev = jnp.concatenate([halo, odd[:, :-1, :]], axis=1)

    def mm(w, rows):
        return jax.lax.dot_general(
            w, rows.reshape(_IN_CH, _RB * _W),
            (((1,), (0,)), ((), ())),
            preferred_element_type=jnp.float32)

    w = wref[...]
    t0 = mm(w[0, 0], oprev) + mm(w[1, 0], even) + mm(w[2, 0], odd)
    t1 = mm(w[0, 1], oprev) + mm(w[1, 1], even) + mm(w[2, 1], odd)
    t2 = mm(w[0, 2], oprev) + mm(w[1, 2], even) + mm(w[2, 2], odd)
    t0 = t0.reshape(_OCB, _RB, _W)
    t1 = t1.reshape(_OCB, _RB, _W)
    t2 = t2.reshape(_OCB, _RB, _W)
    rolled = jnp.concatenate(
        [jnp.zeros((_OCB, _RB, 2), jnp.float32), t0[:, :, :-2]], axis=2)
    u = t2 + rolled
    cat = jnp.concatenate([t1, u], axis=2).reshape(_OCB * _RB, 2 * _W)
    src = jax.lax.broadcasted_iota(jnp.int32, (2 * _W, _WO), 0)
    dst = jax.lax.broadcasted_iota(jnp.int32, (2 * _W, _WO), 1)
    sel = ((src == 2 * dst) | (src == 2 * dst + 1 + _W)).astype(jnp.float32)
    oref[...] = jax.lax.dot_general(
        cat, sel, (((1,), (0,)), ((), ())),
        preferred_element_type=jnp.float32).reshape(_OCB, _RB, _WO)


def _conv_pallas(wt, x):
    return pl.pallas_call(
        _conv_body,
        grid=(_B, _NRB, _NOCB),
        in_specs=[
            pl.BlockSpec((_K, _K, _OCB, _IN_CH), lambda b, k, o: (0, 0, o, 0)),
            pl.BlockSpec((None, _IN_CH, _IRB, _W), lambda b, k, o: (b, 0, k, 0)),
            pl.BlockSpec((None, _IN_CH, 8, _W),
                         lambda b, k, o: (b, 0, jnp.maximum(_IRB // 8 * k - 1, 0), 0)),
        ],
        out_specs=pl.BlockSpec((None, _OCB, _RB, _WO), lambda b, k, o: (b, o, k, 0)),
        out_shape=jax.ShapeDtypeStruct((_B, _PCH, _HO, _WO), jnp.float32),
        compiler_params=pltpu.CompilerParams(
            dimension_semantics=("parallel", "arbitrary", "arbitrary")),
    )(wt, x, x)


def _active_mask(x, weight, hash_a, hash_b):
    w_flat = weight.reshape(_OUT_CH, -1)
    denom = jnp.linalg.norm(w_flat, axis=1).max()
    w_u = _U * w_flat / denom
    norms = jnp.linalg.norm(w_u, axis=1, keepdims=True)
    powers = jnp.concatenate([norms ** (2 ** (i + 1)) for i in range(_M)], axis=1)
    halves = jnp.full((_OUT_CH, _M), 0.5, dtype=w_u.dtype)
    w_pq = jnp.concatenate([w_u, powers, halves], axis=1)
    k_proj = w_pq @ hash_a.T + hash_b[None, :]
    k_idx = jnp.abs(jnp.mod(jnp.floor(k_proj / _R).astype(jnp.int32), _TABLE))

    x_u = _U * x / denom
    q_chan = jnp.full((_B, 1, _H, _W), 0.5, dtype=x.dtype)
    p_chan = jnp.broadcast_to(
        (jnp.linalg.norm(x_u.reshape(_B, -1), axis=1) ** 2).reshape(_B, 1, 1, 1),
        (_B, 1, _H, _W)).astype(x.dtype)
    x_aug = jnp.concatenate([x_u, q_chan, p_chan], axis=1).astype(jnp.bfloat16)
    hk = hash_a.reshape(_NH, _IN_CH + 2, _K, _K).astype(jnp.bfloat16)
    dotted = jax.lax.conv_general_dilated(
        x_aug, hk, window_strides=(_STRIDE, _STRIDE),
        padding=((_PAD, _PAD), (_PAD, _PAD)),
        rhs_dilation=(1, 1),
        dimension_numbers=('NCHW', 'OIHW', 'NCHW'),
        preferred_element_type=jnp.float32)
    bucket = jnp.abs(jnp.mod(
        jnp.floor((dotted + hash_b.reshape(1, -1, 1, 1)) / _R).astype(jnp.int32),
        _TABLE))
    counts = jnp.stack([jnp.bincount(bucket[:, h].ravel(), length=_TABLE)
                        for h in range(_NH)])
    best = jnp.argmax(counts, axis=1)
    return jnp.any(k_idx == best[None, :], axis=1)


def kernel(x, weight, hash_a, hash_b):
    active = _active_mask(x, weight, hash_a, hash_b)
    scale = jnp.asarray(_NH / _TABLE, dtype=x.dtype)
    m = active.astype(x.dtype) * scale
    anchor = jax.lax.conv_general_dilated(
        x, weight[:_ANCH], window_strides=(_STRIDE, _STRIDE),
        padding=((_PAD, _PAD), (_PAD, _PAD)),
        rhs_dilation=(1, 1), dimension_numbers=('NCHW', 'OIHW', 'NCHW'))
    anchor = anchor * m[:_ANCH][None, :, None, None]
    w_eff = weight[_ANCH:] * m[_ANCH:][:, None, None, None]
    wt = jnp.transpose(w_eff, (2, 3, 0, 1))
    rest = _conv_pallas(wt, x)
    return jnp.concatenate([anchor, rest], axis=1)


# speedup vs baseline: 3.7618x; 3.7599x over previous
import jax
import jax.numpy as jnp
from jax.experimental import pallas as pl
from jax.experimental.pallas import tpu as pltpu

_IN_CH = 96
_OUT_CH = 192
_K = 3
_STRIDE = 2
_PAD = 1
_TABLE = 16
_NH = 4
_M = 9
_U = 0.99
_R = 2.5
_B, _H, _W = 2, 224, 224
_HO = _WO = 112
_RB = 16
_NRB = _HO // _RB
_IRB = 2 * _RB
_PCH = _OUT_CH
_OCB = 64
_NOCB = _PCH // _OCB


def _conv_body(wref, xm, xh, oref):
    k = pl.program_id(1)

    xblk = xm[...]
    even = jnp.concatenate([xblk[:, 2 * j:2 * j + 1, :] for j in range(_RB)],
                           axis=1)
    odd = jnp.concatenate([xblk[:, 2 * j + 1:2 * j + 2, :] for j in range(_RB)],
                          axis=1)
    halo = xh[:, 7:8, :]
    halo = jnp.where(k == 0, jnp.zeros_like(halo), halo)
    oprev = jnp.concatenate([halo, odd[:, :-1, :]], axis=1)

    def mm(w, rows):
        return jax.lax.dot_general(
            w, rows.reshape(_IN_CH, _RB * _W),
            (((1,), (0,)), ((), ())),
            preferred_element_type=jnp.float32)

    w = wref[...]
    t0 = mm(w[0, 0], oprev) + mm(w[1, 0], even) + mm(w[2, 0], odd)
    t1 = mm(w[0, 1], oprev) + mm(w[1, 1], even) + mm(w[2, 1], odd)
    t2 = mm(w[0, 2], oprev) + mm(w[1, 2], even) + mm(w[2, 2], odd)
    t0 = t0.reshape(_OCB, _RB, _W)
    t1 = t1.reshape(_OCB, _RB, _W)
    t2 = t2.reshape(_OCB, _RB, _W)
    rolled = jnp.concatenate(
        [jnp.zeros((_OCB, _RB, 2), jnp.float32), t0[:, :, :-2]], axis=2)
    u = t2 + rolled
    cat = jnp.concatenate([t1, u], axis=2).reshape(_OCB * _RB, 2 * _W)
    src = jax.lax.broadcasted_iota(jnp.int32, (2 * _W, _WO), 0)
    dst = jax.lax.broadcasted_iota(jnp.int32, (2 * _W, _WO), 1)
    sel = ((src == 2 * dst) | (src == 2 * dst + 1 + _W)).astype(jnp.float32)
    oref[...] = jax.lax.dot_general(
        cat, sel, (((1,), (0,)), ((), ())),
        preferred_element_type=jnp.float32).reshape(_OCB, _RB, _WO)


def _conv_pallas(wt, x):
    return pl.pallas_call(
        _conv_body,
        grid=(_B, _NRB, _NOCB),
        in_specs=[
            pl.BlockSpec((_K, _K, _OCB, _IN_CH), lambda b, k, o: (0, 0, o, 0)),
            pl.BlockSpec((None, _IN_CH, _IRB, _W), lambda b, k, o: (b, 0, k, 0)),
            pl.BlockSpec((None, _IN_CH, 8, _W),
                         lambda b, k, o: (b, 0, jnp.maximum(_IRB // 8 * k - 1, 0), 0)),
        ],
        out_specs=pl.BlockSpec((None, _OCB, _RB, _WO), lambda b, k, o: (b, o, k, 0)),
        out_shape=jax.ShapeDtypeStruct((_B, _PCH, _HO, _WO), jnp.float32),
        compiler_params=pltpu.CompilerParams(
            dimension_semantics=("parallel", "arbitrary", "arbitrary")),
    )(wt, x, x)


def _aug_body(xref, sref, oref):
    den = sref[0]
    for b in range(_B):
        v = (xref[b] * _U) / den
        t = jnp.transpose(v, (1, 2, 0)).astype(jnp.bfloat16)
        q = jnp.full((8, _W, 1), 0.5, jnp.bfloat16)
        sp = jnp.full((8, _W, 1), sref[1 + b].astype(jnp.bfloat16))
        oref[:, :, b, :] = jnp.concatenate([t, q, sp], axis=-1)


def _aug_pallas(x, scal):
    return pl.pallas_call(
        _aug_body,
        grid=(_H // 8,),
        in_specs=[
            pl.BlockSpec((_B, _IN_CH, 8, _W), lambda k: (0, 0, k, 0)),
            pl.BlockSpec(memory_space=pltpu.SMEM),
        ],
        out_specs=pl.BlockSpec((8, _W, _B, _IN_CH + 2), lambda k: (k, 0, 0, 0)),
        out_shape=jax.ShapeDtypeStruct((_H, _W, _B, _IN_CH + 2), jnp.bfloat16),
    )(x, scal)


def _active_mask(x, weight, hash_a, hash_b):
    w_flat = weight.reshape(_OUT_CH, -1)
    denom = jnp.linalg.norm(w_flat, axis=1).max()
    w_u = _U * w_flat / denom
    norms = jnp.linalg.norm(w_u, axis=1, keepdims=True)
    powers = jnp.concatenate([norms ** (2 ** (i + 1)) for i in range(_M)], axis=1)
    halves = jnp.full((_OUT_CH, _M), 0.5, dtype=w_u.dtype)
    w_pq = jnp.concatenate([w_u, powers, halves], axis=1)
    k_proj = w_pq @ hash_a.T + hash_b[None, :]
    k_idx = jnp.abs(jnp.mod(jnp.floor(k_proj / _R).astype(jnp.int32), _TABLE))

    x_u = _U * x / denom
    s = jnp.linalg.norm(x_u.reshape(_B, -1), axis=1) ** 2
    scal = jnp.concatenate([denom[None], s]).astype(jnp.float32)
    pt = _aug_pallas(x, scal)
    x_aug = jnp.transpose(pt, (2, 3, 0, 1))
    hk = hash_a.reshape(_NH, _IN_CH + 2, _K, _K).astype(jnp.bfloat16)
    dotted = jax.lax.conv_general_dilated(
        x_aug, hk, window_strides=(_STRIDE, _STRIDE),
        padding=((_PAD, _PAD), (_PAD, _PAD)),
        rhs_dilation=(1, 1),
        dimension_numbers=('NCHW', 'OIHW', 'NCHW'),
        preferred_element_type=jnp.float32)
    bucket = jnp.abs(jnp.mod(
        jnp.floor((dotted + hash_b.reshape(1, -1, 1, 1)) / _R).astype(jnp.int32),
        _TABLE))
    counts = jnp.stack([jnp.bincount(bucket[:, h].ravel(), length=_TABLE)
                        for h in range(_NH)])
    best = jnp.argmax(counts, axis=1)
    return jnp.any(k_idx == best[None, :], axis=1)


def kernel(x, weight, hash_a, hash_b):
    active = _active_mask(x, weight, hash_a, hash_b)
    scale = jnp.asarray(_NH / _TABLE, dtype=x.dtype)
    w_eff = weight * (active.astype(x.dtype) * scale)[:, None, None, None]
    wt = jnp.transpose(w_eff, (2, 3, 0, 1))
    return _conv_pallas(wt, x)


# speedup vs baseline: 3.7618x; 1.0000x over previous
import jax
import jax.numpy as jnp
from jax.experimental import pallas as pl
from jax.experimental.pallas import tpu as pltpu

_IN_CH = 96
_OUT_CH = 192
_K = 3
_STRIDE = 2
_PAD = 1
_TABLE = 16
_NH = 4
_M = 9
_U = 0.99
_R = 2.5
_B, _H, _W = 2, 224, 224
_HO = _WO = 112
_RB = 16
_NRB = _HO // _RB
_IRB = 2 * _RB
_PCH = _OUT_CH
_OCB = 64
_NOCB = _PCH // _OCB


def _conv_body(wref, xm, xh, oref):
    k = pl.program_id(1)

    xblk = xm[...]
    even = jnp.concatenate([xblk[:, 2 * j:2 * j + 1, :] for j in range(_RB)],
                           axis=1)
    odd = jnp.concatenate([xblk[:, 2 * j + 1:2 * j + 2, :] for j in range(_RB)],
                          axis=1)
    halo = xh[:, 7:8, :]
    halo = jnp.where(k == 0, jnp.zeros_like(halo), halo)
    oprev = jnp.concatenate([halo, odd[:, :-1, :]], axis=1)

    def mm(w, rows):
        return jax.lax.dot_general(
            w, rows.reshape(_IN_CH, _RB * _W).astype(jnp.bfloat16),
            (((1,), (0,)), ((), ())),
            preferred_element_type=jnp.float32)

    w = wref[...]
    t0 = mm(w[0, 0], oprev) + mm(w[1, 0], even) + mm(w[2, 0], odd)
    t1 = mm(w[0, 1], oprev) + mm(w[1, 1], even) + mm(w[2, 1], odd)
    t2 = mm(w[0, 2], oprev) + mm(w[1, 2], even) + mm(w[2, 2], odd)
    t0 = t0.reshape(_OCB, _RB, _W)
    t1 = t1.reshape(_OCB, _RB, _W)
    t2 = t2.reshape(_OCB, _RB, _W)
    rolled = jnp.concatenate(
        [jnp.zeros((_OCB, _RB, 2), jnp.float32), t0[:, :, :-2]], axis=2)
    u = t2 + rolled
    cat = jnp.concatenate([t1, u], axis=2).reshape(_OCB * _RB, 2 * _W)
    src = jax.lax.broadcasted_iota(jnp.int32, (2 * _W, _WO), 0)
    dst = jax.lax.broadcasted_iota(jnp.int32, (2 * _W, _WO), 1)
    sel = ((src == 2 * dst) | (src == 2 * dst + 1 + _W)).astype(jnp.float32)
    oref[...] = jax.lax.dot_general(
        cat, sel, (((1,), (0,)), ((), ())),
        preferred_element_type=jnp.float32).reshape(_OCB, _RB, _WO)


def _conv_pallas(wt, x):
    return pl.pallas_call(
        _conv_body,
        grid=(_B, _NRB, _NOCB),
        in_specs=[
            pl.BlockSpec((_K, _K, _OCB, _IN_CH), lambda b, k, o: (0, 0, o, 0)),
            pl.BlockSpec((None, _IN_CH, _IRB, _W), lambda b, k, o: (b, 0, k, 0)),
            pl.BlockSpec((None, _IN_CH, 8, _W),
                         lambda b, k, o: (b, 0, jnp.maximum(_IRB // 8 * k - 1, 0), 0)),
        ],
        out_specs=pl.BlockSpec((None, _OCB, _RB, _WO), lambda b, k, o: (b, o, k, 0)),
        out_shape=jax.ShapeDtypeStruct((_B, _PCH, _HO, _WO), jnp.float32),
        compiler_params=pltpu.CompilerParams(
            dimension_semantics=("parallel", "arbitrary", "arbitrary")),
    )(wt, x, x)


def _aug_body(xref, sref, oref):
    den = sref[0]
    for b in range(_B):
        v = (xref[b] * _U) / den
        t = jnp.transpose(v, (1, 2, 0)).astype(jnp.bfloat16)
        q = jnp.full((8, _W, 1), 0.5, jnp.bfloat16)
        sp = jnp.full((8, _W, 1), sref[1 + b].astype(jnp.bfloat16))
        oref[:, :, b, :] = jnp.concatenate([t, q, sp], axis=-1)


def _aug_pallas(x, scal):
    return pl.pallas_call(
        _aug_body,
        grid=(_H // 8,),
        in_specs=[
            pl.BlockSpec((_B, _IN_CH, 8, _W), lambda k: (0, 0, k, 0)),
            pl.BlockSpec(memory_space=pltpu.SMEM),
        ],
        out_specs=pl.BlockSpec((8, _W, _B, _IN_CH + 2), lambda k: (k, 0, 0, 0)),
        out_shape=jax.ShapeDtypeStruct((_H, _W, _B, _IN_CH + 2), jnp.bfloat16),
    )(x, scal)


def _active_mask(x, weight, hash_a, hash_b):
    w_flat = weight.reshape(_OUT_CH, -1)
    denom = jnp.linalg.norm(w_flat, axis=1).max()
    w_u = _U * w_flat / denom
    norms = jnp.linalg.norm(w_u, axis=1, keepdims=True)
    powers = jnp.concatenate([norms ** (2 ** (i + 1)) for i in range(_M)], axis=1)
    halves = jnp.full((_OUT_CH, _M), 0.5, dtype=w_u.dtype)
    w_pq = jnp.concatenate([w_u, powers, halves], axis=1)
    k_proj = w_pq @ hash_a.T + hash_b[None, :]
    k_idx = jnp.abs(jnp.mod(jnp.floor(k_proj / _R).astype(jnp.int32), _TABLE))

    x_u = _U * x / denom
    s = jnp.linalg.norm(x_u.reshape(_B, -1), axis=1) ** 2
    scal = jnp.concatenate([denom[None], s]).astype(jnp.float32)
    pt = _aug_pallas(x, scal)
    x_aug = jnp.transpose(pt, (2, 3, 0, 1))
    hk = hash_a.reshape(_NH, _IN_CH + 2, _K, _K).astype(jnp.bfloat16)
    dotted = jax.lax.conv_general_dilated(
        x_aug, hk, window_strides=(_STRIDE, _STRIDE),
        padding=((_PAD, _PAD), (_PAD, _PAD)),
        rhs_dilation=(1, 1),
        dimension_numbers=('NCHW', 'OIHW', 'NCHW'),
        preferred_element_type=jnp.float32)
    bucket = jnp.abs(jnp.mod(
        jnp.floor((dotted + hash_b.reshape(1, -1, 1, 1)) / _R).astype(jnp.int32),
        _TABLE))
    counts = jnp.stack([jnp.bincount(bucket[:, h].ravel(), length=_TABLE)
                        for h in range(_NH)])
    best = jnp.argmax(counts, axis=1)
    return jnp.any(k_idx == best[None, :], axis=1)


def kernel(x, weight, hash_a, hash_b):
    active = _active_mask(x, weight, hash_a, hash_b)
    scale = jnp.asarray(_NH / _TABLE, dtype=x.dtype)
    w_eff = weight * (active.astype(x.dtype) * scale)[:, None, None, None]
    wt = jnp.transpose(w_eff, (2, 3, 0, 1)).astype(jnp.bfloat16)
    return _conv_pallas(wt, x)


# speedup vs baseline: 6.3534x; 1.6889x over previous
import jax
import jax.numpy as jnp
from jax.experimental import pallas as pl
from jax.experimental.pallas import tpu as pltpu

_IN_CH = 96
_OUT_CH = 192
_K = 3
_STRIDE = 2
_PAD = 1
_TABLE = 16
_NH = 4
_M = 9
_U = 0.99
_R = 2.5
_B, _H, _W = 2, 224, 224
_HO = _WO = 112
_RB = 16
_NRB = _HO // _RB
_IRB = 2 * _RB
_PCH = _OUT_CH
_OCB = 64
_NOCB = _PCH // _OCB


def _conv_body(wref, xm, xh, oref):
    k = pl.program_id(1)

    xblk = xm[...]
    even = jnp.concatenate([xblk[:, 2 * j:2 * j + 1, :] for j in range(_RB)],
                           axis=1)
    odd = jnp.concatenate([xblk[:, 2 * j + 1:2 * j + 2, :] for j in range(_RB)],
                          axis=1)
    halo = xh[:, 7:8, :]
    halo = jnp.where(k == 0, jnp.zeros_like(halo), halo)
    oprev = jnp.concatenate([halo, odd[:, :-1, :]], axis=1)

    def mm(w, rows):
        return jax.lax.dot_general(
            w, rows.reshape(_IN_CH, _RB * _W).astype(jnp.bfloat16),
            (((1,), (0,)), ((), ())),
            preferred_element_type=jnp.float32)

    w = wref[...]
    t0 = mm(w[0, 0], oprev) + mm(w[1, 0], even) + mm(w[2, 0], odd)
    t1 = mm(w[0, 1], oprev) + mm(w[1, 1], even) + mm(w[2, 1], odd)
    t2 = mm(w[0, 2], oprev) + mm(w[1, 2], even) + mm(w[2, 2], odd)
    t0 = t0.reshape(_OCB, _RB, _W)
    t1 = t1.reshape(_OCB, _RB, _W)
    t2 = t2.reshape(_OCB, _RB, _W)
    rolled = jnp.concatenate(
        [jnp.zeros((_OCB, _RB, 2), jnp.float32), t0[:, :, :-2]], axis=2)
    u = t2 + rolled
    cat = jnp.concatenate([t1, u], axis=2).reshape(_OCB * _RB, 2 * _W)
    src = jax.lax.broadcasted_iota(jnp.int32, (2 * _W, _WO), 0)
    dst = jax.lax.broadcasted_iota(jnp.int32, (2 * _W, _WO), 1)
    sel = ((src == 2 * dst) | (src == 2 * dst + 1 + _W)).astype(jnp.float32)
    oref[...] = jax.lax.dot_general(
        cat, sel, (((1,), (0,)), ((), ())),
        preferred_element_type=jnp.float32).reshape(_OCB, _RB, _WO)


def _conv_pallas(wt, x):
    return pl.pallas_call(
        _conv_body,
        grid=(_B, _NRB, _NOCB),
        in_specs=[
            pl.BlockSpec((_K, _K, _OCB, _IN_CH), lambda b, k, o: (0, 0, o, 0)),
            pl.BlockSpec((None, _IN_CH, _IRB, _W), lambda b, k, o: (b, 0, k, 0)),
            pl.BlockSpec((None, _IN_CH, 8, _W),
                         lambda b, k, o: (b, 0, jnp.maximum(_IRB // 8 * k - 1, 0), 0)),
        ],
        out_specs=pl.BlockSpec((None, _OCB, _RB, _WO), lambda b, k, o: (b, o, k, 0)),
        out_shape=jax.ShapeDtypeStruct((_B, _PCH, _HO, _WO), jnp.float32),
        compiler_params=pltpu.CompilerParams(
            dimension_semantics=("parallel", "arbitrary", "arbitrary")),
    )(wt, x, x)


def _aug_body(xref, sref, oref):
    den = sref[0]
    eye = jnp.eye(_IN_CH, dtype=jnp.bfloat16)
    for b in range(_B):
        vb = ((xref[b] * _U) / den).astype(jnp.bfloat16)
        rows = []
        for r in range(8):
            t = jax.lax.dot_general(
                vb[:, r, :], eye, (((0,), (0,)), ((), ())),
                preferred_element_type=jnp.float32)
            rows.append(t.astype(jnp.bfloat16)[None])
        t8 = jnp.concatenate(rows, axis=0)
        q = jnp.full((8, _W, 1), 0.5, jnp.bfloat16)
        sp = jnp.full((8, _W, 1), sref[1 + b].astype(jnp.bfloat16))
        oref[:, :, b, :] = jnp.concatenate([t8, q, sp], axis=-1)


def _aug_pallas(x, scal):
    return pl.pallas_call(
        _aug_body,
        grid=(_H // 8,),
        in_specs=[
            pl.BlockSpec((_B, _IN_CH, 8, _W), lambda k: (0, 0, k, 0)),
            pl.BlockSpec(memory_space=pltpu.SMEM),
        ],
        out_specs=pl.BlockSpec((8, _W, _B, _IN_CH + 2), lambda k: (k, 0, 0, 0)),
        out_shape=jax.ShapeDtypeStruct((_H, _W, _B, _IN_CH + 2), jnp.bfloat16),
    )(x, scal)


def _active_mask(x, weight, hash_a, hash_b):
    w_flat = weight.reshape(_OUT_CH, -1)
    denom = jnp.linalg.norm(w_flat, axis=1).max()
    w_u = _U * w_flat / denom
    norms = jnp.linalg.norm(w_u, axis=1, keepdims=True)
    powers = jnp.concatenate([norms ** (2 ** (i + 1)) for i in range(_M)], axis=1)
    halves = jnp.full((_OUT_CH, _M), 0.5, dtype=w_u.dtype)
    w_pq = jnp.concatenate([w_u, powers, halves], axis=1)
    k_proj = w_pq @ hash_a.T + hash_b[None, :]
    k_idx = jnp.abs(jnp.mod(jnp.floor(k_proj / _R).astype(jnp.int32), _TABLE))

    x_u = _U * x / denom
    s = jnp.linalg.norm(x_u.reshape(_B, -1), axis=1) ** 2
    scal = jnp.concatenate([denom[None], s]).astype(jnp.float32)
    pt = _aug_pallas(x, scal)
    x_aug = jnp.transpose(pt, (2, 3, 0, 1))
    hk = hash_a.reshape(_NH, _IN_CH + 2, _K, _K).astype(jnp.bfloat16)
    dotted = jax.lax.conv_general_dilated(
        x_aug, hk, window_strides=(_STRIDE, _STRIDE),
        padding=((_PAD, _PAD), (_PAD, _PAD)),
        rhs_dilation=(1, 1),
        dimension_numbers=('NCHW', 'OIHW', 'NCHW'),
        preferred_element_type=jnp.float32)
    bucket = jnp.abs(jnp.mod(
        jnp.floor((dotted + hash_b.reshape(1, -1, 1, 1)) / _R).astype(jnp.int32),
        _TABLE))
    counts = jnp.stack([jnp.bincount(bucket[:, h].ravel(), length=_TABLE)
                        for h in range(_NH)])
    best = jnp.argmax(counts, axis=1)
    return jnp.any(k_idx == best[None, :], axis=1)


def kernel(x, weight, hash_a, hash_b):
    active = _active_mask(x, weight, hash_a, hash_b)
    scale = jnp.asarray(_NH / _TABLE, dtype=x.dtype)
    w_eff = weight * (active.astype(x.dtype) * scale)[:, None, None, None]
    wt = jnp.transpose(w_eff, (2, 3, 0, 1)).astype(jnp.bfloat16)
    return _conv_pallas(wt, x)
